# baseline (speedup 1.0000x reference)
import jax
import jax.numpy as jnp
from jax.experimental import pallas as pl
from jax.experimental.pallas import tpu as pltpu

D = 1024
DFF = 2048
E = 8
B = 1024
LOOPS = 8
BLK = 128
KMAX = 8


def _stage_a(x_ref, wc_ref, bc_ref, wd_ref, bd_ref, wr_ref, br_ref,
             ctx_ref, p0_ref, p1_ref, v0_ref, v1_ref, poff_ref, nblk_ref):
    x = x_ref[...]
    comp = jnp.dot(x, wc_ref[...], preferred_element_type=jnp.float32) + bc_ref[...]
    ctx = jnp.dot(comp, wd_ref[...], preferred_element_type=jnp.float32) + bd_ref[...]
    ctx_ref[...] = ctx

    logits = jnp.dot(ctx, wr_ref[...], preferred_element_type=jnp.float32) + br_ref[...]
    m = jnp.max(logits, axis=-1, keepdims=True)
    ex = jnp.exp(logits - m)
    probs = ex / jnp.sum(ex, axis=-1, keepdims=True)
    lane8 = jax.lax.broadcasted_iota(jnp.int32, probs.shape, 1)
    v1 = jnp.max(probs, axis=-1, keepdims=True)
    i1 = jnp.argmax(probs, axis=-1)[:, None]
    masked = jnp.where(lane8 == i1, -jnp.inf, probs)
    v2 = jnp.max(masked, axis=-1, keepdims=True)
    i2 = jnp.argmax(masked, axis=-1)[:, None]
    s = v1 + v2
    v0_ref[...] = v1 / s
    v1_ref[...] = v2 / s

    oh0 = jnp.where(lane8 == i1, 1.0, 0.0)
    oh1 = jnp.where(lane8 == i2, 1.0, 0.0)
    T = oh0.shape[0]
    rr = jax.lax.broadcasted_iota(jnp.int32, (T, T), 0)
    cc = jax.lax.broadcasted_iota(jnp.int32, (T, T), 1)
    tril = jnp.where(rr >= cc, 1.0, 0.0)
    c0 = jnp.dot(tril, oh0, preferred_element_type=jnp.float32)
    c1 = jnp.dot(tril, oh1, preferred_element_type=jnp.float32)
    counts0 = c0[T - 1:T, :]

    ones_t = jnp.full((T, 1), 1.0, jnp.float32)
    counts_s = jax.lax.dot_general(
        oh0 + oh1, ones_t, (((0,), (0,)), ((), ())),
        preferred_element_type=jnp.float32)
    nblk_s = jnp.floor((counts_s + (BLK - 1)) * (1.0 / BLK))
    padded_s = nblk_s * BLK
    r8 = jax.lax.broadcasted_iota(jnp.int32, (E, E), 0)
    c8 = jax.lax.broadcasted_iota(jnp.int32, (E, E), 1)
    tril8 = jnp.where(r8 >= c8, 1.0, 0.0)
    ends_s = jnp.dot(tril8, padded_s, preferred_element_type=jnp.float32)
    poff_s = ends_s - padded_s
    poff_ref[...] = poff_s.astype(jnp.int32)
    nblk_ref[...] = nblk_s.astype(jnp.int32)

    rank0 = jnp.sum(oh0 * (c0 - 1.0), axis=1, keepdims=True)
    rank1 = jnp.sum(oh1 * (counts0 + c1 - 1.0), axis=1, keepdims=True)
    base0 = jnp.dot(oh0, poff_s, preferred_element_type=jnp.float32)
    base1 = jnp.dot(oh1, poff_s, preferred_element_type=jnp.float32)
    p0_ref[...] = (base0 + rank0).astype(jnp.int32)
    p1_ref[...] = (base1 + rank1).astype(jnp.int32)


def _stage_bc(poff_ref, nblk_ref, ctx_ref, w1_ref, b1_ref, w2_ref, b2_ref,
              p0_ref, p1_ref, v0_ref, v1_ref,
              wc_ref, bcr_ref, wh1_ref, bh1_ref, wh2_ref, bh2_ref,
              out_ref, acc_ref):
    e = pl.program_id(0)

    @pl.when(e == 0)
    def _():
        acc_ref[...] = ctx_ref[...]

    @pl.when(e < E)
    def _():
        em = jnp.minimum(e, E - 1)
        base = poff_ref[em]
        nblk = nblk_ref[em]
        p0 = p0_ref[...]
        p1 = p1_ref[...]
        v0 = v0_ref[...]
        v1 = v1_ref[...]
        for k in range(KMAX):
            @pl.when(k < nblk)
            def _():
                slane = (jax.lax.broadcasted_iota(jnp.int32, (B, BLK), 1)
                         + base + k * BLK)
                cmp0 = slane == p0
                cmp1 = slane == p1
                gt = jnp.where(cmp0 | cmp1, 1.0, 0.0)
                xg = jax.lax.dot_general(
                    gt, ctx_ref[...], (((0,), (0,)), ((), ())),
                    preferred_element_type=jnp.float32)
                h = jnp.dot(xg.astype(jnp.bfloat16), w1_ref[0],
                            preferred_element_type=jnp.float32) + b1_ref[0]
                h = jnp.maximum(h, 0.0)
                y = jnp.dot(h.astype(jnp.bfloat16), w2_ref[0],
                            preferred_element_type=jnp.float32) + b2_ref[0]
                mw = (jnp.where(cmp0, 1.0, 0.0) * v0
                      + jnp.where(cmp1, 1.0, 0.0) * v1)
                acc_ref[...] += jnp.dot(mw, y,
                                        preferred_element_type=jnp.float32)

    @pl.when(e == E)
    def _():
        wc = wc_ref[...]
        bc = bcr_ref[...]

        def body(_, h):
            dh = jnp.tanh(jnp.dot(h, wc, preferred_element_type=jnp.float32) + bc) - h
            return h + 0.1 * dh

        h = jax.lax.fori_loop(0, LOOPS, body, acc_ref[...])
        hidden = jnp.dot(h, wh1_ref[...], preferred_element_type=jnp.float32) + bh1_ref[...]
        hidden = jnp.maximum(hidden, 0.0)
        out_ref[...] = jnp.dot(hidden, wh2_ref[...], preferred_element_type=jnp.float32) + bh2_ref[...]


def kernel(x, W_comp, b_comp, W_db, b_db, W_router, b_router, W1, b1, W2, b2,
           W_core, b_core, W_h1, b_h1, W_h2, b_h2):
    T = x.shape[0] * x.shape[1]
    xt = x.reshape(T, D)

    ctx, p0, p1, v0, v1, poff, nblk = pl.pallas_call(
        _stage_a,
        out_shape=(
            jax.ShapeDtypeStruct((T, D), jnp.float32),
            jax.ShapeDtypeStruct((T, 1), jnp.int32),
            jax.ShapeDtypeStruct((T, 1), jnp.int32),
            jax.ShapeDtypeStruct((T, 1), jnp.float32),
            jax.ShapeDtypeStruct((T, 1), jnp.float32),
            jax.ShapeDtypeStruct((E, 1), jnp.int32),
            jax.ShapeDtypeStruct((E, 1), jnp.int32),
        ),
    )(xt, W_comp, b_comp.reshape(1, D), W_db, b_db.reshape(1, D),
      W_router, b_router.reshape(1, E))

    em = lambda e, poff, nblk: (jnp.minimum(e, E - 1), 0, 0)
    out = pl.pallas_call(
        _stage_bc,
        grid_spec=pltpu.PrefetchScalarGridSpec(
            num_scalar_prefetch=2,
            grid=(E + 1,),
            in_specs=[
                pl.BlockSpec((T, D), lambda e, poff, nblk: (0, 0)),
                pl.BlockSpec((1, D, DFF), em),
                pl.BlockSpec((1, 1, DFF), em),
                pl.BlockSpec((1, DFF, D), em),
                pl.BlockSpec((1, 1, D), em),
                pl.BlockSpec((T, 1), lambda e, poff, nblk: (0, 0)),
                pl.BlockSpec((T, 1), lambda e, poff, nblk: (0, 0)),
                pl.BlockSpec((T, 1), lambda e, poff, nblk: (0, 0)),
                pl.BlockSpec((T, 1), lambda e, poff, nblk: (0, 0)),
                pl.BlockSpec((D, D), lambda e, poff, nblk: (0, 0)),
                pl.BlockSpec((1, D), lambda e, poff, nblk: (0, 0)),
                pl.BlockSpec((D, 256), lambda e, poff, nblk: (0, 0)),
                pl.BlockSpec((1, 256), lambda e, poff, nblk: (0, 0)),
                pl.BlockSpec((256, 1), lambda e, poff, nblk: (0, 0)),
                pl.BlockSpec((1, 1), lambda e, poff, nblk: (0, 0)),
            ],
            out_specs=pl.BlockSpec((T, 1), lambda e, poff, nblk: (0, 0)),
            scratch_shapes=[pltpu.VMEM((T, D), jnp.float32)],
        ),
        out_shape=jax.ShapeDtypeStruct((T, 1), jnp.float32),
        compiler_params=pltpu.CompilerParams(
            vmem_limit_bytes=100 * 1024 * 1024),
    )(poff.reshape(E), nblk.reshape(E),
      ctx, W1.astype(jnp.bfloat16), b1.reshape(E, 1, DFF),
      W2.astype(jnp.bfloat16), b2.reshape(E, 1, D),
      p0, p1, v0, v1,
      W_core, b_core.reshape(1, D), W_h1, b_h1.reshape(1, 256),
      W_h2, b_h2.reshape(1, 1))

    return out


# speedup vs baseline: 1.4845x; 1.4845x over previous
import jax
import jax.numpy as jnp
from jax.experimental import pallas as pl
from jax.experimental.pallas import tpu as pltpu

D = 1024
DFF = 2048
E = 8
B = 1024
LOOPS = 8
BLK = 128
KMAX = 8


def _stage_a(x_ref, wc_ref, bc_ref, wd_ref, bd_ref, wr_ref, br_ref,
             ctx_ref, p0_ref, p1_ref, v0_ref, v1_ref, poff_ref, nblk_ref):
    x = x_ref[...]
    comp = jnp.dot(x, wc_ref[...], preferred_element_type=jnp.float32) + bc_ref[...]
    ctx = jnp.dot(comp, wd_ref[...], preferred_element_type=jnp.float32) + bd_ref[...]
    ctx_ref[...] = ctx

    logits = jnp.dot(ctx, wr_ref[...], preferred_element_type=jnp.float32) + br_ref[...]
    m = jnp.max(logits, axis=-1, keepdims=True)
    ex = jnp.exp(logits - m)
    probs = ex / jnp.sum(ex, axis=-1, keepdims=True)
    lane8 = jax.lax.broadcasted_iota(jnp.int32, probs.shape, 1)
    v1 = jnp.max(probs, axis=-1, keepdims=True)
    i1 = jnp.argmax(probs, axis=-1)[:, None]
    masked = jnp.where(lane8 == i1, -jnp.inf, probs)
    v2 = jnp.max(masked, axis=-1, keepdims=True)
    i2 = jnp.argmax(masked, axis=-1)[:, None]
    s = v1 + v2
    v0_ref[...] = v1 / s
    v1_ref[...] = v2 / s

    oh0 = jnp.where(lane8 == i1, 1.0, 0.0)
    oh1 = jnp.where(lane8 == i2, 1.0, 0.0)
    T = oh0.shape[0]
    rr = jax.lax.broadcasted_iota(jnp.int32, (T, T), 0)
    cc = jax.lax.broadcasted_iota(jnp.int32, (T, T), 1)
    tril = jnp.where(rr >= cc, 1.0, 0.0)
    c0 = jnp.dot(tril, oh0, preferred_element_type=jnp.float32)
    c1 = jnp.dot(tril, oh1, preferred_element_type=jnp.float32)
    counts0 = c0[T - 1:T, :]

    ones_t = jnp.full((T, 1), 1.0, jnp.float32)
    counts_s = jax.lax.dot_general(
        oh0 + oh1, ones_t, (((0,), (0,)), ((), ())),
        preferred_element_type=jnp.float32)
    nblk_s = jnp.floor((counts_s + (BLK - 1)) * (1.0 / BLK))
    padded_s = nblk_s * BLK
    r8 = jax.lax.broadcasted_iota(jnp.int32, (E, E), 0)
    c8 = jax.lax.broadcasted_iota(jnp.int32, (E, E), 1)
    tril8 = jnp.where(r8 >= c8, 1.0, 0.0)
    ends_s = jnp.dot(tril8, padded_s, preferred_element_type=jnp.float32)
    poff_s = ends_s - padded_s
    poff_ref[...] = poff_s.astype(jnp.int32)
    nblk_ref[...] = nblk_s.astype(jnp.int32)

    rank0 = jnp.sum(oh0 * (c0 - 1.0), axis=1, keepdims=True)
    rank1 = jnp.sum(oh1 * (counts0 + c1 - 1.0), axis=1, keepdims=True)
    base0 = jnp.dot(oh0, poff_s, preferred_element_type=jnp.float32)
    base1 = jnp.dot(oh1, poff_s, preferred_element_type=jnp.float32)
    p0_ref[...] = (base0 + rank0).astype(jnp.int32)
    p1_ref[...] = (base1 + rank1).astype(jnp.int32)


def _stage_bc(poff_ref, nblk_ref, ctx_ref, w1_ref, b1_ref, w2_ref, b2_ref,
              p0_ref, p1_ref, v0_ref, v1_ref,
              wc_ref, bcr_ref, wh1_ref, bh1_ref, wh2_ref, bh2_ref,
              out_ref, acc_ref):
    e = pl.program_id(0)

    @pl.when(e == 0)
    def _():
        acc_ref[...] = ctx_ref[...]

    @pl.when(e < E)
    def _():
        em = jnp.minimum(e, E - 1)
        base = poff_ref[em]
        nblk = nblk_ref[em]
        p0 = p0_ref[...]
        p1 = p1_ref[...]
        v0 = v0_ref[...]
        v1 = v1_ref[...]
        for k in range(KMAX):
            @pl.when(k < nblk)
            def _():
                slane = (jax.lax.broadcasted_iota(jnp.int32, (B, BLK), 1)
                         + base + k * BLK)
                cmp0 = slane == p0
                cmp1 = slane == p1
                gt = jnp.where(cmp0 | cmp1, 1.0, 0.0)
                xg = jax.lax.dot_general(
                    gt, ctx_ref[...], (((0,), (0,)), ((), ())),
                    preferred_element_type=jnp.float32)
                h = jnp.dot(xg, w1_ref[0],
                            preferred_element_type=jnp.float32) + b1_ref[0]
                h = jnp.maximum(h, 0.0)
                y = jnp.dot(h, w2_ref[0],
                            preferred_element_type=jnp.float32) + b2_ref[0]
                mw = (jnp.where(cmp0, 1.0, 0.0) * v0
                      + jnp.where(cmp1, 1.0, 0.0) * v1)
                acc_ref[...] += jnp.dot(mw, y,
                                        preferred_element_type=jnp.float32)

    @pl.when(e == E)
    def _():
        wc = wc_ref[...].astype(jnp.bfloat16)
        bc = bcr_ref[...]

        def body(_, h):
            z = jnp.dot(h.astype(jnp.bfloat16), wc,
                        preferred_element_type=jnp.float32) + bc
            return h + 0.1 * (jnp.tanh(z) - h)

        h = jax.lax.fori_loop(0, LOOPS, body, acc_ref[...])
        hidden = jnp.dot(h, wh1_ref[...], preferred_element_type=jnp.float32) + bh1_ref[...]
        hidden = jnp.maximum(hidden, 0.0)
        out_ref[...] = jnp.dot(hidden, wh2_ref[...], preferred_element_type=jnp.float32) + bh2_ref[...]


def kernel(x, W_comp, b_comp, W_db, b_db, W_router, b_router, W1, b1, W2, b2,
           W_core, b_core, W_h1, b_h1, W_h2, b_h2):
    T = x.shape[0] * x.shape[1]
    xt = x.reshape(T, D)

    ctx, p0, p1, v0, v1, poff, nblk = pl.pallas_call(
        _stage_a,
        out_shape=(
            jax.ShapeDtypeStruct((T, D), jnp.float32),
            jax.ShapeDtypeStruct((T, 1), jnp.int32),
            jax.ShapeDtypeStruct((T, 1), jnp.int32),
            jax.ShapeDtypeStruct((T, 1), jnp.float32),
            jax.ShapeDtypeStruct((T, 1), jnp.float32),
            jax.ShapeDtypeStruct((E, 1), jnp.int32),
            jax.ShapeDtypeStruct((E, 1), jnp.int32),
        ),
    )(xt, W_comp, b_comp.reshape(1, D), W_db, b_db.reshape(1, D),
      W_router, b_router.reshape(1, E))

    em = lambda e, poff, nblk: (jnp.minimum(e, E - 1), 0, 0)
    out = pl.pallas_call(
        _stage_bc,
        grid_spec=pltpu.PrefetchScalarGridSpec(
            num_scalar_prefetch=2,
            grid=(E + 1,),
            in_specs=[
                pl.BlockSpec((T, D), lambda e, poff, nblk: (0, 0)),
                pl.BlockSpec((1, D, DFF), em),
                pl.BlockSpec((1, 1, DFF), em),
                pl.BlockSpec((1, DFF, D), em),
                pl.BlockSpec((1, 1, D), em),
                pl.BlockSpec((T, 1), lambda e, poff, nblk: (0, 0)),
                pl.BlockSpec((T, 1), lambda e, poff, nblk: (0, 0)),
                pl.BlockSpec((T, 1), lambda e, poff, nblk: (0, 0)),
                pl.BlockSpec((T, 1), lambda e, poff, nblk: (0, 0)),
                pl.BlockSpec((D, D), lambda e, poff, nblk: (0, 0)),
                pl.BlockSpec((1, D), lambda e, poff, nblk: (0, 0)),
                pl.BlockSpec((D, 256), lambda e, poff, nblk: (0, 0)),
                pl.BlockSpec((1, 256), lambda e, poff, nblk: (0, 0)),
                pl.BlockSpec((256, 1), lambda e, poff, nblk: (0, 0)),
                pl.BlockSpec((1, 1), lambda e, poff, nblk: (0, 0)),
            ],
            out_specs=pl.BlockSpec((T, 1), lambda e, poff, nblk: (0, 0)),
            scratch_shapes=[pltpu.VMEM((T, D), jnp.float32)],
        ),
        out_shape=jax.ShapeDtypeStruct((T, 1), jnp.float32),
        compiler_params=pltpu.CompilerParams(
            vmem_limit_bytes=100 * 1024 * 1024),
    )(poff.reshape(E), nblk.reshape(E),
      ctx, W1, b1.reshape(E, 1, DFF), W2, b2.reshape(E, 1, D),
      p0, p1, v0, v1,
      W_core, b_core.reshape(1, D), W_h1, b_h1.reshape(1, 256),
      W_h2, b_h2.reshape(1, 1))

    return out
